# Initial kernel scaffold; baseline (speedup 1.0000x reference)
import jax
import jax.numpy as jnp
from jax.experimental import pallas as pl


def kernel(x, table):
    raise NotImplementedError("write your pallas kernel here")


# speedup vs baseline: 1.6853x; 1.6853x over previous
import functools

import jax
import jax.numpy as jnp
from jax import lax
from jax.experimental import pallas as pl
from jax.experimental.pallas import tpu as pltpu
from jax.experimental.pallas import tpu_sc as plsc

_BATCH, _SEQ = 16384, 50
_D = 64
_B = _BATCH * _SEQ
_NC, _NS = 2, 16
_NW = _NC * _NS
_BPW = _B // _NW
_CHUNK = 128
_NCHUNK = _BPW // _CHUNK

_mesh = plsc.VectorSubcoreMesh(core_axis_name="c", subcore_axis_name="s")


@functools.partial(
    pl.kernel,
    mesh=_mesh,
    out_type=jax.ShapeDtypeStruct((_B, _D), jnp.float32),
    scratch_types=[
        pltpu.VMEM((_NCHUNK, _CHUNK), jnp.int32),
        pltpu.VMEM((_CHUNK, _D), jnp.float32),
        pltpu.SemaphoreType.DMA,
    ],
    compiler_params=pltpu.CompilerParams(use_tc_tiling_on_sc=False),
)
def _embed(x_hbm, table_hbm, out_hbm, idx_v, rows_v, sem):
    wid = lax.axis_index("s") * _NC + lax.axis_index("c")
    base = wid * _BPW
    pltpu.sync_copy(x_hbm.at[wid], idx_v)

    def step(j, carry):
        pltpu.async_copy(table_hbm.at[idx_v.at[j]], rows_v, sem).wait()
        pltpu.sync_copy(rows_v, out_hbm.at[pl.ds(base + j * _CHUNK, _CHUNK)])
        return carry

    lax.fori_loop(0, _NCHUNK, step, 0)


def kernel(x, table):
    xi = x.reshape(_NW, _NCHUNK, _CHUNK).astype(jnp.int32)
    out = _embed(xi, table)
    return out.reshape(_BATCH, _SEQ, _D)


# speedup vs baseline: 1.8831x; 1.1174x over previous
import functools

import jax
import jax.numpy as jnp
from jax import lax
from jax.experimental import pallas as pl
from jax.experimental.pallas import tpu as pltpu
from jax.experimental.pallas import tpu_sc as plsc

_BATCH, _SEQ = 16384, 50
_D = 64
_B = _BATCH * _SEQ
_NC, _NS = 2, 16
_NW = _NC * _NS
_BPW = _B // _NW
_CHUNK = 128
_NCHUNK = _BPW // _CHUNK
_CPB = 4
_BLK = _CHUNK * _CPB
_NBLK = _BPW // _BLK

_mesh = plsc.VectorSubcoreMesh(core_axis_name="c", subcore_axis_name="s")


@functools.partial(
    pl.kernel,
    mesh=_mesh,
    out_type=jax.ShapeDtypeStruct((_B, _D), jnp.float32),
    scratch_types=[
        pltpu.VMEM((_NCHUNK, _CHUNK), jnp.int32),
        pltpu.VMEM((_BLK, _D), jnp.float32),
        pltpu.VMEM((_BLK, _D), jnp.float32),
        pltpu.SemaphoreType.DMA,
        pltpu.SemaphoreType.DMA,
        pltpu.SemaphoreType.DMA,
        pltpu.SemaphoreType.DMA,
    ],
    compiler_params=pltpu.CompilerParams(use_tc_tiling_on_sc=False),
)
def _embed(x_hbm, table_hbm, out_hbm, idx_v, rows0, rows1, g0, g1, s0, s1):
    wid = lax.axis_index("s") * _NC + lax.axis_index("c")
    base = wid * _BPW
    rows = (rows0, rows1)
    gsem = (g0, g1)
    ssem = (s0, s1)

    pltpu.sync_copy(x_hbm.at[wid], idx_v)

    def fire_gathers(j, b):
        for k in range(_CPB):
            pltpu.async_copy(
                table_hbm.at[idx_v.at[j * _CPB + k]],
                rows[b].at[pl.ds(k * _CHUNK, _CHUNK)],
                gsem[b],
            )

    def wait_gathers(j, b):
        for k in range(_CPB):
            pltpu.make_async_copy(
                table_hbm.at[idx_v.at[j * _CPB + k]],
                rows[b].at[pl.ds(k * _CHUNK, _CHUNK)],
                gsem[b],
            ).wait()

    def store_desc(j, b):
        return pltpu.make_async_copy(
            rows[b], out_hbm.at[pl.ds(base + j * _BLK, _BLK)], ssem[b])

    fire_gathers(0, 0)

    def pair(g2, carry):
        for b in range(2):
            j = g2 * 2 + b
            wait_gathers(j, b)
            store_desc(j, b).start()
            @pl.when(j >= 1)
            def _():
                store_desc(j - 1, 1 - b).wait()

            @pl.when(j + 1 < _NBLK)
            def _():
                fire_gathers(j + 1, 1 - b)
        return carry

    lax.fori_loop(0, _NBLK // 2, pair, 0)
    store_desc(_NBLK - 1, 1).wait()


def kernel(x, table):
    xi = x.reshape(_NW, _NCHUNK, _CHUNK).astype(jnp.int32)
    out = _embed(xi, table)
    return out.reshape(_BATCH, _SEQ, _D)
